# baseline (device time: 36603 ns/iter reference)
import jax
import jax.numpy as jnp
from jax import lax
from jax.experimental import pallas as pl
from jax.experimental.pallas import tpu as pltpu

N_LAYERS = 3
N_CHUNKS = 2


def kernel(x, Win0, Wout0, Win1, Wout1, Win2, Wout2):
    b, d_in = x.shape
    _, h_blk = Win0.shape
    C = h_blk // N_CHUNKS

    def body(x_ref, win0_ref, wout0_ref, win1_ref, wout1_ref,
             win2_ref, wout2_ref, out_ref,
             sendh, sendx, recvh, recvx, hs_sems, hr_sems, xs_sems, xr_sems):
        my_x = lax.axis_index("x")
        my_y = lax.axis_index("y")
        y_peer = (my_x, 1 - my_y)
        x_peer = (1 - my_x, my_y)

        barrier_sem = pltpu.get_barrier_semaphore()
        for nbr in (y_peer, x_peer):
            pl.semaphore_signal(
                barrier_sem, inc=1,
                device_id=nbr, device_id_type=pl.DeviceIdType.MESH,
            )
        pl.semaphore_wait(barrier_sem, 2)

        for l in range(N_LAYERS):
            rdmas = []
            for c in range(N_CHUNKS):
                sendh[l, c] = x_ref[:, :]
                rd = pltpu.make_async_remote_copy(
                    src_ref=sendh.at[l, c],
                    dst_ref=recvh.at[l, c],
                    send_sem=hs_sems.at[l, c],
                    recv_sem=hr_sems.at[l, c],
                    device_id=y_peer,
                    device_id_type=pl.DeviceIdType.MESH,
                )
                rd.start()
                rdmas.append(rd)
            for c in range(N_CHUNKS):
                rdmas[c].wait_recv()
                rdmas[c].wait_send()
            sendx[l] = recvh[l, 0]
            xrd = pltpu.make_async_remote_copy(
                src_ref=sendx.at[l],
                dst_ref=recvx.at[l],
                send_sem=xs_sems.at[l],
                recv_sem=xr_sems.at[l],
                device_id=x_peer,
                device_id_type=pl.DeviceIdType.MESH,
            )
            xrd.start()
            xrd.wait_recv()
            xrd.wait_send()
        out_ref[:, :] = recvx[N_LAYERS - 1]

    return pl.pallas_call(
        body,
        out_shape=jax.ShapeDtypeStruct((b, d_in), jnp.float32),
        in_specs=[pl.BlockSpec(memory_space=pltpu.VMEM)] * 7,
        out_specs=pl.BlockSpec(memory_space=pltpu.VMEM),
        scratch_shapes=[
            pltpu.VMEM((N_LAYERS, N_CHUNKS, b, C), jnp.float32),
            pltpu.VMEM((N_LAYERS, b, d_in), jnp.float32),
            pltpu.VMEM((N_LAYERS, N_CHUNKS, b, C), jnp.float32),
            pltpu.VMEM((N_LAYERS, b, d_in), jnp.float32),
            pltpu.SemaphoreType.DMA((N_LAYERS, N_CHUNKS)),
            pltpu.SemaphoreType.DMA((N_LAYERS, N_CHUNKS)),
            pltpu.SemaphoreType.DMA((N_LAYERS,)),
            pltpu.SemaphoreType.DMA((N_LAYERS,)),
        ],
        compiler_params=pltpu.CompilerParams(collective_id=0),
    )(x, Win0, Wout0, Win1, Wout1, Win2, Wout2)


# device time: 33126 ns/iter; 1.1050x vs baseline; 1.1050x over previous
import jax
import jax.numpy as jnp
from jax import lax
from jax.experimental import pallas as pl
from jax.experimental.pallas import tpu as pltpu

N_LAYERS = 3


def kernel(x, Win0, Wout0, Win1, Wout1, Win2, Wout2):
    b, d_in = x.shape
    _, h_blk = Win0.shape

    def body(x_ref, win0_ref, wout0_ref, win1_ref, wout1_ref,
             win2_ref, wout2_ref, out_ref,
             w_in, w_out, sendh, sendx, recvh, recvx,
             w_sems, hs_sems, hr_sems, xs_sems, xr_sems):
        my_x = lax.axis_index("x")
        my_y = lax.axis_index("y")
        y_peer = (my_x, 1 - my_y)
        x_peer = (1 - my_x, my_y)

        win_hbm = (win0_ref, win1_ref, win2_ref)
        wout_hbm = (wout0_ref, wout1_ref, wout2_ref)
        w_copies = []
        for l in range(N_LAYERS):
            cin = pltpu.make_async_copy(win_hbm[l], w_in.at[l], w_sems.at[l, 0])
            cin.start()
            cout = pltpu.make_async_copy(wout_hbm[l], w_out.at[l], w_sems.at[l, 1])
            cout.start()
            w_copies.append((cin, cout))

        barrier_sem = pltpu.get_barrier_semaphore()
        for nbr in (y_peer, x_peer):
            pl.semaphore_signal(
                barrier_sem, inc=1,
                device_id=nbr, device_id_type=pl.DeviceIdType.MESH,
            )
        pl.semaphore_wait(barrier_sem, 2)

        xcur = x_ref[:, :]
        for l in range(N_LAYERS):
            w_copies[l][0].wait()
            ph = jnp.dot(xcur, w_in[l], preferred_element_type=jnp.float32)
            sendh[l] = ph.astype(jnp.bfloat16)
            rdma = pltpu.make_async_remote_copy(
                src_ref=sendh.at[l],
                dst_ref=recvh.at[l],
                send_sem=hs_sems.at[l],
                recv_sem=hr_sems.at[l],
                device_id=y_peer,
                device_id_type=pl.DeviceIdType.MESH,
            )
            rdma.start()
            w_copies[l][1].wait()
            rdma.wait()
            h = jnp.maximum(ph + recvh[l].astype(jnp.float32), 0.0)

            px = jnp.dot(h, w_out[l], preferred_element_type=jnp.float32)
            sendx[l] = px.astype(jnp.bfloat16)
            rdma2 = pltpu.make_async_remote_copy(
                src_ref=sendx.at[l],
                dst_ref=recvx.at[l],
                send_sem=xs_sems.at[l],
                recv_sem=xr_sems.at[l],
                device_id=x_peer,
                device_id_type=pl.DeviceIdType.MESH,
            )
            rdma2.start()
            rdma2.wait()
            xcur = px + recvx[l].astype(jnp.float32)

        out_ref[:, :] = xcur

    return pl.pallas_call(
        body,
        out_shape=jax.ShapeDtypeStruct((b, d_in), jnp.float32),
        in_specs=[pl.BlockSpec(memory_space=pltpu.VMEM)]
        + [pl.BlockSpec(memory_space=pl.ANY)] * 6,
        out_specs=pl.BlockSpec(memory_space=pltpu.VMEM),
        scratch_shapes=[
            pltpu.VMEM((N_LAYERS, d_in, h_blk), jnp.float32),
            pltpu.VMEM((N_LAYERS, h_blk, d_in), jnp.float32),
            pltpu.VMEM((N_LAYERS, b, h_blk), jnp.bfloat16),
            pltpu.VMEM((N_LAYERS, b, d_in), jnp.bfloat16),
            pltpu.VMEM((N_LAYERS, b, h_blk), jnp.bfloat16),
            pltpu.VMEM((N_LAYERS, b, d_in), jnp.bfloat16),
            pltpu.SemaphoreType.DMA((N_LAYERS, 2)),
            pltpu.SemaphoreType.DMA((N_LAYERS,)),
            pltpu.SemaphoreType.DMA((N_LAYERS,)),
            pltpu.SemaphoreType.DMA((N_LAYERS,)),
            pltpu.SemaphoreType.DMA((N_LAYERS,)),
        ],
        compiler_params=pltpu.CompilerParams(collective_id=0),
    )(x, Win0, Wout0, Win1, Wout1, Win2, Wout2)


# device time: 33119 ns/iter; 1.1052x vs baseline; 1.0002x over previous
import jax
import jax.numpy as jnp
from jax import lax
from jax.experimental import pallas as pl
from jax.experimental.pallas import tpu as pltpu

N_LAYERS = 3


def kernel(x, Win0, Wout0, Win1, Wout1, Win2, Wout2):
    b, d_in = x.shape
    _, h_blk = Win0.shape

    def body(x_ref, win0_ref, wout0_ref, win1_ref, wout1_ref,
             win2_ref, wout2_ref, out_ref,
             w_in, w_out, w_in_bf, w_out_bf, sendh, sendx, recvh, recvx,
             w_sems, hs_sems, hr_sems, xs_sems, xr_sems):
        my_x = lax.axis_index("x")
        my_y = lax.axis_index("y")
        y_peer = (my_x, 1 - my_y)
        x_peer = (1 - my_x, my_y)

        win_hbm = (win0_ref, win1_ref, win2_ref)
        wout_hbm = (wout0_ref, wout1_ref, wout2_ref)
        w_copies = []
        for l in range(N_LAYERS):
            cin = pltpu.make_async_copy(win_hbm[l], w_in.at[l], w_sems.at[l, 0])
            cin.start()
            cout = pltpu.make_async_copy(wout_hbm[l], w_out.at[l], w_sems.at[l, 1])
            cout.start()
            w_copies.append((cin, cout))

        barrier_sem = pltpu.get_barrier_semaphore()
        for nbr in (y_peer, x_peer):
            pl.semaphore_signal(
                barrier_sem, inc=1,
                device_id=nbr, device_id_type=pl.DeviceIdType.MESH,
            )
        pl.semaphore_wait(barrier_sem, 2)

        in_cast = [False] * N_LAYERS
        out_cast = [False] * N_LAYERS

        def ensure_in(l):
            if not in_cast[l]:
                w_copies[l][0].wait()
                w_in_bf[l] = w_in[l].astype(jnp.bfloat16)
                in_cast[l] = True

        def ensure_out(l):
            if not out_cast[l]:
                w_copies[l][1].wait()
                w_out_bf[l] = w_out[l].astype(jnp.bfloat16)
                out_cast[l] = True

        xcur = x_ref[:, :].astype(jnp.bfloat16)
        for l in range(N_LAYERS):
            ensure_in(l)
            ph = jnp.dot(xcur, w_in_bf[l], preferred_element_type=jnp.float32)
            sendh[l] = ph.astype(jnp.bfloat16)
            rdma = pltpu.make_async_remote_copy(
                src_ref=sendh.at[l],
                dst_ref=recvh.at[l],
                send_sem=hs_sems.at[l],
                recv_sem=hr_sems.at[l],
                device_id=y_peer,
                device_id_type=pl.DeviceIdType.MESH,
            )
            rdma.start()
            ensure_out(l)
            rdma.wait()
            h = jnp.maximum(ph + recvh[l].astype(jnp.float32), 0.0)
            h_bf = h.astype(jnp.bfloat16)

            px = jnp.dot(h_bf, w_out_bf[l], preferred_element_type=jnp.float32)
            sendx[l] = px.astype(jnp.bfloat16)
            rdma2 = pltpu.make_async_remote_copy(
                src_ref=sendx.at[l],
                dst_ref=recvx.at[l],
                send_sem=xs_sems.at[l],
                recv_sem=xr_sems.at[l],
                device_id=x_peer,
                device_id_type=pl.DeviceIdType.MESH,
            )
            rdma2.start()
            if l + 1 < N_LAYERS:
                ensure_in(l + 1)
            rdma2.wait()
            xcur = (px + recvx[l].astype(jnp.float32)).astype(jnp.bfloat16)
            if l + 1 == N_LAYERS:
                out_ref[:, :] = px + recvx[l].astype(jnp.float32)

    return pl.pallas_call(
        body,
        out_shape=jax.ShapeDtypeStruct((b, d_in), jnp.float32),
        in_specs=[pl.BlockSpec(memory_space=pltpu.VMEM)]
        + [pl.BlockSpec(memory_space=pl.ANY)] * 6,
        out_specs=pl.BlockSpec(memory_space=pltpu.VMEM),
        scratch_shapes=[
            pltpu.VMEM((N_LAYERS, d_in, h_blk), jnp.float32),
            pltpu.VMEM((N_LAYERS, h_blk, d_in), jnp.float32),
            pltpu.VMEM((N_LAYERS, d_in, h_blk), jnp.bfloat16),
            pltpu.VMEM((N_LAYERS, h_blk, d_in), jnp.bfloat16),
            pltpu.VMEM((N_LAYERS, b, h_blk), jnp.bfloat16),
            pltpu.VMEM((N_LAYERS, b, d_in), jnp.bfloat16),
            pltpu.VMEM((N_LAYERS, b, h_blk), jnp.bfloat16),
            pltpu.VMEM((N_LAYERS, b, d_in), jnp.bfloat16),
            pltpu.SemaphoreType.DMA((N_LAYERS, 2)),
            pltpu.SemaphoreType.DMA((N_LAYERS,)),
            pltpu.SemaphoreType.DMA((N_LAYERS,)),
            pltpu.SemaphoreType.DMA((N_LAYERS,)),
            pltpu.SemaphoreType.DMA((N_LAYERS,)),
        ],
        compiler_params=pltpu.CompilerParams(collective_id=0),
    )(x, Win0, Wout0, Win1, Wout1, Win2, Wout2)
